# baseline (speedup 1.0000x reference)
import functools
import math

import jax
import jax.numpy as jnp
from jax.experimental import pallas as pl
from jax.experimental.pallas import tpu as pltpu

DIM = 128
KV_LEN = 8192
BB = 2
NSPLIT = 4
CHUNK = KV_LEN // NSPLIT


def _attn_kernel(q_ref, *refs):
    k_refs = refs[:NSPLIT]
    v_refs = refs[NSPLIT:2 * NSPLIT]
    ctx_ref, attn_ref = refs[2 * NSPLIT], refs[2 * NSPLIT + 1]
    scale = 1.0 / math.sqrt(DIM)
    for i in range(BB):
        q = q_ref[i]
        ss = [
            jax.lax.dot_general(
                q, kr[i], (((1,), (1,)), ((), ())),
                preferred_element_type=jnp.float32,
            ) * scale
            for kr in k_refs
        ]
        m = functools.reduce(jnp.maximum, [jnp.max(s) for s in ss])
        ps = [jnp.exp(s - m) for s in ss]
        denom = functools.reduce(jnp.add, [jnp.sum(p) for p in ps])
        inv = 1.0 / denom
        ctx = jnp.zeros((1, DIM), jnp.float32)
        for c, (p, vr) in enumerate(zip(ps, v_refs)):
            a = p * inv
            ctx = ctx + jnp.dot(a, vr[i], preferred_element_type=jnp.float32)
            attn_ref[i, :, c * CHUNK:(c + 1) * CHUNK] = a
        ctx_ref[i] = ctx


@jax.jit
def kernel(query, key, value):
    batch, q_len, dim = query.shape
    kv_len = key.shape[1]
    chunk = kv_len // NSPLIT
    grid = (batch // BB,)
    out_ctx = jax.ShapeDtypeStruct((batch, q_len, dim), jnp.float32)
    out_attn = jax.ShapeDtypeStruct((batch, q_len, kv_len), jnp.float32)

    def _kv_spec(c):
        return pl.BlockSpec((BB, chunk, dim), lambda b, c=c: (b, c, 0))

    ctx, attn = pl.pallas_call(
        _attn_kernel,
        grid=grid,
        in_specs=(
            [pl.BlockSpec((BB, q_len, dim), lambda b: (b, 0, 0))]
            + [_kv_spec(c) for c in range(NSPLIT)]
            + [_kv_spec(c) for c in range(NSPLIT)]
        ),
        out_specs=[
            pl.BlockSpec((BB, q_len, dim), lambda b: (b, 0, 0)),
            pl.BlockSpec((BB, q_len, kv_len), lambda b: (b, 0, 0)),
        ],
        out_shape=[out_ctx, out_attn],
        compiler_params=pltpu.CompilerParams(
            dimension_semantics=("parallel",),
        ),
    )(query, *([key] * NSPLIT), *([value] * NSPLIT))
    return (ctx, attn)
